# Initial kernel scaffold; baseline (speedup 1.0000x reference)
import jax
import jax.numpy as jnp
from jax.experimental import pallas as pl


def kernel(users, items, user_emb, item_emb, edge_src, edge_dst, edge_w):
    raise NotImplementedError("write your pallas kernel here")


# speedup vs baseline: 2.7108x; 2.7108x over previous
import functools

import jax
import jax.numpy as jnp
from jax import lax
from jax.experimental import pallas as pl
from jax.experimental.pallas import tpu as pltpu
from jax.experimental.pallas import tpu_sc as plsc

N_USERS = 15000
N_ITEMS = 35000
NN = N_USERS + N_ITEMS
E = 800000
D = 64
NL = 4
B = 4096

NC = 2
NS = 16
LANES = 16
NQ = D // LANES
QPC = NQ // NC
DH = D // NC
EPT = E // NS
CH = 2000
NCHUNK = EPT // CH
RPT = NN // NS
BPT = B // NS

_mesh = plsc.VectorSubcoreMesh(core_axis_name="c", subcore_axis_name="s")


@functools.partial(
    pl.kernel,
    out_type=jax.ShapeDtypeStruct((NC, B), jnp.float32),
    mesh=_mesh,
    compiler_params=pltpu.CompilerParams(needs_layout_passes=False,
                                         use_tc_tiling_on_sc=False),
    scratch_types=[
        pltpu.HBM((NQ * NN, LANES), jnp.float32),
        pltpu.VMEM_SHARED((NN, LANES), jnp.float32),
        pltpu.VMEM((CH, LANES), jnp.float32),
        pltpu.VMEM((CH,), jnp.int32),
        pltpu.VMEM((CH,), jnp.int32),
        pltpu.VMEM((CH,), jnp.float32),
        pltpu.VMEM((BPT, DH), jnp.float32),
        pltpu.VMEM((BPT, DH), jnp.float32),
        pltpu.VMEM((BPT,), jnp.int32),
        pltpu.VMEM((BPT,), jnp.int32),
        pltpu.VMEM((BPT,), jnp.int32),
        pltpu.VMEM((BPT,), jnp.float32),
    ],
)
def _ltocf_sc(users, items, x0, src, dst, w, out,
              xcur, acc, rows, sidx, didx, wbuf, bsu, bsi,
              uidl, iidl, qid, ob):
    c = lax.axis_index("c")
    t = lax.axis_index("s")
    iota = lax.iota(jnp.int32, LANES)
    zv = jnp.zeros((LANES,), jnp.float32)

    pltpu.sync_copy(users.at[pl.ds(t * BPT, BPT)], uidl)
    pltpu.sync_copy(items.at[pl.ds(t * BPT, BPT)], iidl)

    def _prep(i, carry):
        s = pl.ds(i * LANES, LANES)
        iidl[s] = iidl[s] + N_USERS
        return carry
    lax.fori_loop(0, BPT // LANES, _prep, 0)

    def _zb(i, carry):
        s0 = pl.ds(0, LANES)
        s1 = pl.ds(LANES, LANES)
        bsu[i, s0] = zv
        bsu[i, s1] = zv
        bsi[i, s0] = zv
        bsi[i, s1] = zv
        return carry
    lax.fori_loop(0, BPT, _zb, 0)

    def _fold(dst_ref, q):
        s = pl.ds(q * LANES, LANES)

        def f(i, carry):
            dst_ref[i, s] = dst_ref[i, s] + rows[i, pl.ds(0, LANES)]
            return carry
        lax.fori_loop(0, BPT, f, 0)

    def _qoff(idref, qq):
        def f(i, carry):
            s = pl.ds(i * LANES, LANES)
            qid[s] = idref[s] + qq * NN
            return carry
        lax.fori_loop(0, BPT // LANES, f, 0)

    for q in range(QPC):
        qq = c * QPC + q
        _qoff(uidl, qq)
        pltpu.sync_copy(x0.at[qid], rows.at[pl.ds(0, BPT)])
        _fold(bsu, q)
        _qoff(iidl, qq)
        pltpu.sync_copy(x0.at[qid], rows.at[pl.ds(0, BPT)])
        _fold(bsi, q)

    for k in range(NL):
        srctab = x0 if k == 0 else xcur
        for q in range(QPC):
            qq = c * QPC + q
            qbase = qq * NN

            def _zr(i, carry):
                rows[i, pl.ds(0, LANES)] = zv
                return carry
            lax.fori_loop(0, CH, _zr, 0)
            pltpu.sync_copy(rows, acc.at[pl.ds(t * RPT, CH)])
            pltpu.sync_copy(rows.at[pl.ds(0, RPT - CH)],
                            acc.at[pl.ds(t * RPT + CH, RPT - CH)])
            plsc.subcore_barrier()

            def _chunk(j, carry):
                ebase = t * EPT + j * CH
                pltpu.sync_copy(src.at[pl.ds(ebase, CH)], sidx)
                pltpu.sync_copy(dst.at[pl.ds(ebase, CH)], didx)
                pltpu.sync_copy(w.at[pl.ds(ebase, CH)], wbuf)

                def _off(i, cc):
                    s = pl.ds(i * LANES, LANES)
                    sidx[s] = sidx[s] + qbase
                    return cc
                lax.fori_loop(0, CH // LANES, _off, 0)

                pltpu.sync_copy(srctab.at[sidx], rows)

                def _scale(g, cc):
                    rbase = g * LANES
                    wv = wbuf[pl.ds(rbase, LANES)]
                    ridx = iota + rbase
                    for dcol in range(LANES):
                        cidx = jnp.full((LANES,), dcol, jnp.int32)
                        v = plsc.load_gather(rows, [ridx, cidx])
                        plsc.store_scatter(rows, [ridx, cidx], v * wv)
                    return cc
                lax.fori_loop(0, CH // LANES, _scale, 0)

                pltpu.sync_copy(rows, acc.at[didx], add=True)
                return carry
            lax.fori_loop(0, NCHUNK, _chunk, 0)
            plsc.subcore_barrier()

            pltpu.sync_copy(acc.at[uidl], rows.at[pl.ds(0, BPT)])
            _fold(bsu, q)
            pltpu.sync_copy(acc.at[iidl], rows.at[pl.ds(0, BPT)])
            _fold(bsi, q)

            if k < NL - 1:
                pltpu.sync_copy(acc.at[pl.ds(t * RPT, RPT)],
                                xcur.at[pl.ds(qbase + t * RPT, RPT)])
            plsc.subcore_barrier()

    def _dot(g, carry):
        gb = g * LANES
        ridx = iota + gb
        accv = jnp.zeros((LANES,), jnp.float32)
        for dcol in range(DH):
            cidx = jnp.full((LANES,), dcol, jnp.int32)
            uv = plsc.load_gather(bsu, [ridx, cidx])
            iv = plsc.load_gather(bsi, [ridx, cidx])
            accv = accv + uv * iv
        ob[pl.ds(gb, LANES)] = accv * (1.0 / ((NL + 1) * (NL + 1)))
        return carry
    lax.fori_loop(0, BPT // LANES, _dot, 0)

    pltpu.sync_copy(ob, out.at[c, pl.ds(t * BPT, BPT)])


def kernel(users, items, user_emb, item_emb, edge_src, edge_dst, edge_w):
    all_emb = jnp.concatenate([user_emb, item_emb], axis=0)
    xq = all_emb.reshape(NN, NQ, LANES).transpose(1, 0, 2).reshape(NQ * NN, LANES)
    part = _ltocf_sc(users, items, xq, edge_src, edge_dst, edge_w)
    return part[0] + part[1]
